# Initial kernel scaffold; baseline (speedup 1.0000x reference)
import jax
import jax.numpy as jnp
from jax.experimental import pallas as pl


def kernel(queries, keys):
    raise NotImplementedError("write your pallas kernel here")


# speedup vs baseline: 3.6056x; 3.6056x over previous
import jax
import jax.numpy as jnp
from jax.experimental import pallas as pl
from jax.experimental.pallas import tpu as pltpu

N_DOCS = 5
Q = 1024
D = 128
K = 100000
BK = 2048
NK = (K + BK - 1) // BK
KPAD = NK * BK
CHUNK = 128
NCH = BK // CHUNK

NEG_INF = float("-inf")
IMAX = jnp.iinfo(jnp.int32).max


def _body(q_ref, k_ref, out_v_ref, out_i_ref, tv_ref, ti_ref):
    kb = pl.program_id(0)

    @pl.when(kb == 0)
    def _init():
        tv_ref[...] = jnp.full(tv_ref.shape, NEG_INF, jnp.float32)
        ti_ref[...] = jnp.zeros(ti_ref.shape, jnp.int32)

    s = jax.lax.dot_general(
        q_ref[...], k_ref[...],
        dimension_numbers=(((1,), (1,)), ((), ())),
        preferred_element_type=jnp.float32,
    )

    base = kb * BK
    col_iota = jax.lax.broadcasted_iota(jnp.int32, (Q, CHUNK), 1)
    for r in range(NCH):
        w = s[:, r * CHUNK:(r + 1) * CHUNK]
        wid = col_iota + (base + r * CHUNK)
        w = jnp.where(wid < K, w, NEG_INF)
        for t in range(N_DOCS):
            tv = tv_ref[t]
            ti = ti_ref[t]
            gt = w > tv
            tv_ref[t] = jnp.maximum(tv, w)
            ti_ref[t] = jnp.where(gt, wid, ti)
            if t < N_DOCS - 1:
                w, wid = jnp.minimum(tv, w), jnp.where(gt, ti, wid)

    @pl.when(kb == NK - 1)
    def _merge():
        cv = jnp.concatenate([tv_ref[t] for t in range(N_DOCS)], axis=1)
        ci = jnp.concatenate([ti_ref[t] for t in range(N_DOCS)], axis=1)
        for t in range(N_DOCS):
            m = jnp.max(cv, axis=1, keepdims=True)
            hit = cv == m
            sel = jnp.min(jnp.where(hit, ci, IMAX), axis=1, keepdims=True)
            out_v_ref[:, pl.ds(t, 1)] = m
            out_i_ref[:, pl.ds(t, 1)] = sel
            cv = jnp.where(hit & (ci == sel), NEG_INF, cv)


def kernel(queries, keys):
    keys_p = jnp.pad(keys, ((0, KPAD - K), (0, 0)))
    out_v, out_i = pl.pallas_call(
        _body,
        grid=(NK,),
        in_specs=[
            pl.BlockSpec((Q, D), lambda k: (0, 0)),
            pl.BlockSpec((BK, D), lambda k: (k, 0)),
        ],
        out_specs=[
            pl.BlockSpec((Q, N_DOCS), lambda k: (0, 0)),
            pl.BlockSpec((Q, N_DOCS), lambda k: (0, 0)),
        ],
        out_shape=[
            jax.ShapeDtypeStruct((Q, N_DOCS), jnp.float32),
            jax.ShapeDtypeStruct((Q, N_DOCS), jnp.int32),
        ],
        scratch_shapes=[
            pltpu.VMEM((N_DOCS, Q, CHUNK), jnp.float32),
            pltpu.VMEM((N_DOCS, Q, CHUNK), jnp.int32),
        ],
        compiler_params=pltpu.CompilerParams(
            dimension_semantics=("arbitrary",),
        ),
    )(queries, keys_p)
    return out_v, out_i
